# Initial kernel scaffold; baseline (speedup 1.0000x reference)
import jax
import jax.numpy as jnp
from jax.experimental import pallas as pl


def kernel(token_ids, token_table, pos_table):
    raise NotImplementedError("write your pallas kernel here")


# speedup vs baseline: 1.2624x; 1.2624x over previous
import functools
import jax
import jax.numpy as jnp
from jax import lax
from jax.experimental import pallas as pl
from jax.experimental.pallas import tpu as pltpu
from jax.experimental.pallas import tpu_sc as plsc

CONTEXT = 2048
EMBED = 128
NBATCH = 4
NC, NS, L = 2, 16, 16
NW = NC * NS
ROWS = NBATCH * CONTEXT
R_PER_W = ROWS // NW
IDX_CHUNK = 128
N_CHUNKS = R_PER_W // IDX_CHUNK


def _sc_embed(token_ids_2d, token_table, pos_table):
    mesh = plsc.VectorSubcoreMesh(core_axis_name="c", subcore_axis_name="s")

    @functools.partial(
        pl.kernel,
        out_type=jax.ShapeDtypeStruct((ROWS, EMBED), jnp.float32),
        mesh=mesh,
        scratch_types=[
            pltpu.VMEM((N_CHUNKS, IDX_CHUNK), jnp.int32),
            pltpu.VMEM((R_PER_W, EMBED), jnp.float32),
            pltpu.VMEM((R_PER_W, EMBED), jnp.float32),
            pltpu.SemaphoreType.DMA,
        ],
    )
    def body(ids_hbm, table_hbm, pos_hbm, out_hbm, idx_v, rows_v, pos_v, sem):
        wid = lax.axis_index("s") * NC + lax.axis_index("c")
        base = wid * R_PER_W
        pltpu.sync_copy(ids_hbm.at[pl.ds(wid * N_CHUNKS, N_CHUNKS)], idx_v)
        pos_base = lax.rem(wid, CONTEXT // R_PER_W) * R_PER_W
        pos_cp = pltpu.make_async_copy(
            pos_hbm.at[pl.ds(pos_base, R_PER_W)], pos_v, sem
        )
        pos_cp.start()
        gathers = [
            pltpu.make_async_copy(
                table_hbm.at[idx_v.at[j]],
                rows_v.at[pl.ds(j * IDX_CHUNK, IDX_CHUNK)],
                sem,
            )
            for j in range(N_CHUNKS)
        ]
        for g in gathers:
            g.start()
        pos_cp.wait()
        for g in gathers:
            g.wait()

        def add_row(i):
            for k in range(EMBED // L):
                sl = pl.ds(k * L, L)
                rows_v[i, sl] = rows_v[i, sl] + pos_v[i, sl]

        lax.fori_loop(0, R_PER_W, lambda i, _: (add_row(i), 0)[1], 0)

        pltpu.sync_copy(rows_v, out_hbm.at[pl.ds(base, R_PER_W)])

    return body(token_ids_2d, token_table, pos_table)


def kernel(token_ids, token_table, pos_table):
    ids_flat = token_ids.astype(jnp.int32).reshape(ROWS // IDX_CHUNK, IDX_CHUNK)
    out = _sc_embed(ids_flat, token_table, pos_table)
    return out.reshape(NBATCH, CONTEXT, EMBED)


# speedup vs baseline: 1.3339x; 1.0566x over previous
import functools
import jax
import jax.numpy as jnp
from jax import lax
from jax.experimental import pallas as pl
from jax.experimental.pallas import tpu as pltpu
from jax.experimental.pallas import tpu_sc as plsc

CONTEXT = 2048
EMBED = 128
NBATCH = 4
NC, NS, L = 2, 16, 16
NW = NC * NS
ROWS = NBATCH * CONTEXT
R_PER_W = ROWS // NW
IDX_CHUNK = 128
N_CHUNKS = R_PER_W // IDX_CHUNK


def _sc_embed(token_ids_2d, token_table, pos_table):
    mesh = plsc.VectorSubcoreMesh(core_axis_name="c", subcore_axis_name="s")

    @functools.partial(
        pl.kernel,
        out_type=jax.ShapeDtypeStruct((ROWS, EMBED), jnp.float32),
        mesh=mesh,
        scratch_types=[
            pltpu.VMEM((N_CHUNKS, IDX_CHUNK), jnp.int32),
            pltpu.VMEM((R_PER_W, EMBED), jnp.float32),
            pltpu.SemaphoreType.DMA,
        ],
    )
    def body(ids_hbm, table_hbm, pos_hbm, out_hbm, idx_v, rows_v, sem):
        wid = lax.axis_index("s") * NC + lax.axis_index("c")
        base = wid * R_PER_W
        pltpu.sync_copy(ids_hbm.at[pl.ds(wid * N_CHUNKS, N_CHUNKS)], idx_v)
        pos_base = lax.rem(wid, CONTEXT // R_PER_W) * R_PER_W
        pltpu.sync_copy(pos_hbm.at[pl.ds(pos_base, R_PER_W)], rows_v)
        gathers = [
            pltpu.async_copy(
                table_hbm.at[idx_v.at[j]],
                rows_v.at[pl.ds(j * IDX_CHUNK, IDX_CHUNK)],
                sem,
                add=True,
            )
            for j in range(N_CHUNKS)
        ]
        for g in gathers:
            g.wait()

        pltpu.sync_copy(rows_v, out_hbm.at[pl.ds(base, R_PER_W)])

    return body(token_ids_2d, token_table, pos_table)


def kernel(token_ids, token_table, pos_table):
    ids_flat = token_ids.astype(jnp.int32).reshape(ROWS // IDX_CHUNK, IDX_CHUNK)
    out = _sc_embed(ids_flat, token_table, pos_table)
    return out.reshape(NBATCH, CONTEXT, EMBED)


# speedup vs baseline: 1.3708x; 1.0276x over previous
import functools
import jax
import jax.numpy as jnp
from jax import lax
from jax.experimental import pallas as pl
from jax.experimental.pallas import tpu as pltpu
from jax.experimental.pallas import tpu_sc as plsc

CONTEXT = 2048
EMBED = 128
NBATCH = 4
NC, NS, L = 2, 16, 16
NW = NC * NS
ROWS = NBATCH * CONTEXT
R_PER_W = ROWS // NW
N_CHUNKS = 4
CHUNK = R_PER_W // N_CHUNKS


def _sc_embed(token_ids_2d, token_table, pos_table):
    mesh = plsc.VectorSubcoreMesh(core_axis_name="c", subcore_axis_name="s")

    @functools.partial(
        pl.kernel,
        out_type=jax.ShapeDtypeStruct((ROWS, EMBED), jnp.float32),
        mesh=mesh,
        scratch_types=[
            pltpu.VMEM((N_CHUNKS, CHUNK), jnp.int32),
            pltpu.VMEM((R_PER_W, EMBED), jnp.float32),
            pltpu.SemaphoreType.DMA,
        ]
        + [pltpu.SemaphoreType.DMA] * N_CHUNKS
        + [pltpu.SemaphoreType.DMA] * N_CHUNKS,
    )
    def body(ids_hbm, table_hbm, pos_hbm, out_hbm, idx_v, rows_v, sem_io, *sems):
        sem_pos = sems[:N_CHUNKS]
        sem_g = sems[N_CHUNKS:]
        wid = lax.axis_index("s") * NC + lax.axis_index("c")
        base = wid * R_PER_W
        pos_base = lax.rem(wid, CONTEXT // R_PER_W) * R_PER_W

        idx_cp = pltpu.async_copy(
            ids_hbm.at[pl.ds(wid * N_CHUNKS, N_CHUNKS)], idx_v, sem_io
        )
        pos_cps = [
            pltpu.async_copy(
                pos_hbm.at[pl.ds(pos_base + j * CHUNK, CHUNK)],
                rows_v.at[pl.ds(j * CHUNK, CHUNK)],
                sem_pos[j],
            )
            for j in range(N_CHUNKS)
        ]
        idx_cp.wait()

        gathers = []
        for j in range(N_CHUNKS):
            pos_cps[j].wait()
            gathers.append(
                pltpu.async_copy(
                    table_hbm.at[idx_v.at[j]],
                    rows_v.at[pl.ds(j * CHUNK, CHUNK)],
                    sem_g[j],
                    add=True,
                )
            )
        writes = []
        for j in range(N_CHUNKS):
            gathers[j].wait()
            writes.append(
                pltpu.async_copy(
                    rows_v.at[pl.ds(j * CHUNK, CHUNK)],
                    out_hbm.at[pl.ds(base + j * CHUNK, CHUNK)],
                    sem_io,
                )
            )
        for w in writes:
            w.wait()

    return body(token_ids_2d, token_table, pos_table)


def kernel(token_ids, token_table, pos_table):
    ids_flat = token_ids.astype(jnp.int32).reshape(ROWS // CHUNK, CHUNK)
    out = _sc_embed(ids_flat, token_table, pos_table)
    return out.reshape(NBATCH, CONTEXT, EMBED)
